# Initial kernel scaffold; baseline (speedup 1.0000x reference)
import jax
import jax.numpy as jnp
from jax.experimental import pallas as pl


def kernel(gates):
    raise NotImplementedError("write your pallas kernel here")


# speedup vs baseline: 6.2907x; 6.2907x over previous
import functools

import jax
import jax.numpy as jnp
from jax import lax
from jax.experimental import pallas as pl
from jax.experimental.pallas import tpu as pltpu
from jax.experimental.pallas import tpu_sc as plsc

B = 32768
E = 64
K = 8
NC, NS = 2, 16
NW = NC * NS
ROWS_PER_W = B // NW
CHUNK = 256
N_CHUNKS = ROWS_PER_W // CHUNK
GROUPS = CHUNK // 16
CAPACITY = int(B * 1.25 / E)


def _router_body(gates_hbm, rw_hbm, tki_hbm, tkw_hbm, pcnt_hbm,
                 gbuf, rwbuf, tkibuf, tkwbuf, cntbuf):
    wid = lax.axis_index("s") * NC + lax.axis_index("c")
    lane = lax.iota(jnp.int32, 16)
    zf = jnp.zeros((16,), jnp.float32)
    zi = jnp.zeros((16,), jnp.int32)

    for q in range(E // 16):
        cntbuf[pl.ds(q * 16, 16)] = zi

    def chunk_body(c, _):
        base = wid * (ROWS_PER_W * E) + c * (CHUNK * E)
        pltpu.sync_copy(gates_hbm.at[pl.ds(base, CHUNK * E)], gbuf)

        def group_body(g, _):
            goff = g * (16 * E)
            for r in range(16):
                for q in range(E // 16):
                    rwbuf[pl.ds(goff + r * E + q * 16, 16)] = zf

            neg = jnp.full((16,), -jnp.inf, jnp.float32)
            m = [neg] * K
            mi = [zi] * K
            col_base = goff + lane * E
            for e in range(E):
                v = plsc.load_gather(gbuf, [col_base + e])
                ei = jnp.full((16,), e, jnp.int32)
                for j in range(K):
                    swap = v > m[j]
                    hi = jnp.maximum(m[j], v)
                    lo = jnp.minimum(m[j], v)
                    hidx = jnp.where(swap, ei, mi[j])
                    lidx = jnp.where(swap, mi[j], ei)
                    m[j], mi[j] = hi, hidx
                    v, ei = lo, lidx

            p = [jnp.exp(m[j] - m[0]) for j in range(K)]
            s = p[0]
            for j in range(1, K):
                s = s + p[j]
            inv = 1.0 / s
            w = [p[j] * inv for j in range(K)]

            trow = g * (16 * K) + lane * K
            one = jnp.ones((16,), jnp.int32)
            for j in range(K):
                plsc.store_scatter(rwbuf, [col_base + mi[j]], w[j])
                plsc.store_scatter(tkwbuf, [trow + j], w[j])
                plsc.store_scatter(tkibuf, [trow + j], mi[j])
                plsc.addupdate_scatter(
                    cntbuf, [mi[j]], jnp.where(w[j] > 0.0, one, zi))
            return 0

        lax.fori_loop(0, GROUPS, group_body, 0)

        pltpu.sync_copy(rwbuf, rw_hbm.at[pl.ds(base, CHUNK * E)])
        tbase = wid * (ROWS_PER_W * K) + c * (CHUNK * K)
        pltpu.sync_copy(tkibuf, tki_hbm.at[pl.ds(tbase, CHUNK * K)])
        pltpu.sync_copy(tkwbuf, tkw_hbm.at[pl.ds(tbase, CHUNK * K)])
        return 0

    lax.fori_loop(0, N_CHUNKS, chunk_body, 0)
    pltpu.sync_copy(cntbuf, pcnt_hbm.at[pl.ds(wid * E, E)])


@functools.partial(
    pl.kernel,
    out_type=(
        jax.ShapeDtypeStruct((B * E,), jnp.float32),
        jax.ShapeDtypeStruct((B * K,), jnp.int32),
        jax.ShapeDtypeStruct((B * K,), jnp.float32),
        jax.ShapeDtypeStruct((NW * E,), jnp.int32),
    ),
    mesh=plsc.VectorSubcoreMesh(core_axis_name="c", subcore_axis_name="s"),
    compiler_params=pltpu.CompilerParams(needs_layout_passes=False),
    scratch_types=[
        pltpu.VMEM((CHUNK * E,), jnp.float32),
        pltpu.VMEM((CHUNK * E,), jnp.float32),
        pltpu.VMEM((CHUNK * K,), jnp.int32),
        pltpu.VMEM((CHUNK * K,), jnp.float32),
        pltpu.VMEM((E,), jnp.int32),
    ],
)
def _router(gates_hbm, rw_hbm, tki_hbm, tkw_hbm, pcnt_hbm,
            gbuf, rwbuf, tkibuf, tkwbuf, cntbuf):
    _router_body(gates_hbm, rw_hbm, tki_hbm, tkw_hbm, pcnt_hbm,
                 gbuf, rwbuf, tkibuf, tkwbuf, cntbuf)


def _stats_body(pc_ref, loads_ref, util_ref, exc_ref):
    pc = pc_ref[...]
    loads = jnp.sum(pc, axis=0, keepdims=True)
    loads_ref[...] = loads
    util_ref[...] = loads.astype(jnp.float32) * (1.0 / B)
    exc_ref[...] = (loads > CAPACITY).astype(jnp.int32)


_stats = pl.pallas_call(
    _stats_body,
    out_shape=(
        jax.ShapeDtypeStruct((1, E), jnp.int32),
        jax.ShapeDtypeStruct((1, E), jnp.float32),
        jax.ShapeDtypeStruct((1, E), jnp.int32),
    ),
)


def kernel(gates):
    rw, tki, tkw, pcnt = _router(gates.reshape(-1))
    loads, util, exc = _stats(pcnt.reshape(NW, E))
    return (
        rw.reshape(B, E),
        tki.reshape(B, K),
        util.reshape(E),
        loads.reshape(E),
        exc.reshape(E).astype(jnp.bool_),
        tkw.reshape(B, K),
    )


# speedup vs baseline: 7.3988x; 1.1761x over previous
import functools

import jax
import jax.numpy as jnp
from jax import lax
from jax.experimental import pallas as pl
from jax.experimental.pallas import tpu as pltpu
from jax.experimental.pallas import tpu_sc as plsc

B = 32768
E = 64
K = 8
NC, NS = 2, 16
NW = NC * NS
ROWS_PER_W = B // NW
CHUNK = 256
N_CHUNKS = ROWS_PER_W // CHUNK
GROUPS = CHUNK // 16
CAPACITY = int(B * 1.25 / E)

_NET19 = [(0, 1), (2, 3), (4, 5), (6, 7),
          (0, 2), (1, 3), (4, 6), (5, 7),
          (1, 2), (5, 6),
          (0, 4), (1, 5), (2, 6), (3, 7),
          (2, 4), (3, 5),
          (1, 2), (3, 4), (5, 6)]
_BITONIC12 = [(0, 4), (1, 5), (2, 6), (3, 7),
              (0, 2), (1, 3), (4, 6), (5, 7),
              (0, 1), (2, 3), (4, 5), (6, 7)]


def _router_body(gates_hbm, rw_hbm, tki_hbm, tkw_hbm, pcnt_hbm,
                 gbuf, rwbuf, tkibuf, tkwbuf, cntbuf):
    wid = lax.axis_index("s") * NC + lax.axis_index("c")
    lane = lax.iota(jnp.int32, 16)
    zf = jnp.zeros((16,), jnp.float32)
    zi = jnp.zeros((16,), jnp.int32)

    for q in range(E // 16):
        cntbuf[pl.ds(q * 16, 16)] = zi

    def chunk_body(c, _):
        base = wid * (ROWS_PER_W * E) + c * (CHUNK * E)
        pltpu.sync_copy(gates_hbm.at[pl.ds(base, CHUNK * E)], gbuf)

        def group_body(g, _):
            goff = g * (16 * E)
            for r in range(16):
                for q in range(E // 16):
                    rwbuf[pl.ds(goff + r * E + q * 16, 16)] = zf

            col_base = goff + lane * E

            def keys_of(t):
                ks = []
                for j in range(8):
                    e = t * 8 + j
                    v = plsc.load_gather(gbuf, [col_base + e])
                    bits = lax.bitcast_convert_type(v, jnp.int32)
                    kb = (bits & -64) | (63 - e)
                    ks.append(lax.bitcast_convert_type(kb, jnp.float32))
                return ks

            def sort8(v):
                for i, j in _NET19:
                    v[i], v[j] = jnp.maximum(v[i], v[j]), jnp.minimum(v[i], v[j])
                return v

            def merge_top8(a, b):
                c = [jnp.maximum(a[i], b[7 - i]) for i in range(8)]
                for i, j in _BITONIC12:
                    c[i], c[j] = jnp.maximum(c[i], c[j]), jnp.minimum(c[i], c[j])
                return c

            m01 = merge_top8(sort8(keys_of(0)), sort8(keys_of(1)))
            m23 = merge_top8(sort8(keys_of(2)), sort8(keys_of(3)))
            m03 = merge_top8(m01, m23)
            m45 = merge_top8(sort8(keys_of(4)), sort8(keys_of(5)))
            m67 = merge_top8(sort8(keys_of(6)), sort8(keys_of(7)))
            m47 = merge_top8(m45, m67)
            f = merge_top8(m03, m47)

            mi = [63 - (lax.bitcast_convert_type(f[j], jnp.int32) & 63)
                  for j in range(K)]
            m = [plsc.load_gather(gbuf, [col_base + mi[j]]) for j in range(K)]

            p = [jnp.exp(m[j] - m[0]) for j in range(K)]
            s = p[0]
            for j in range(1, K):
                s = s + p[j]
            inv = 1.0 / s
            w = [p[j] * inv for j in range(K)]

            trow = g * (16 * K) + lane * K
            one = jnp.ones((16,), jnp.int32)
            for j in range(K):
                plsc.store_scatter(rwbuf, [col_base + mi[j]], w[j])
                plsc.store_scatter(tkwbuf, [trow + j], w[j])
                plsc.store_scatter(tkibuf, [trow + j], mi[j])
                plsc.addupdate_scatter(
                    cntbuf, [mi[j]], jnp.where(w[j] > 0.0, one, zi))
            return 0

        lax.fori_loop(0, GROUPS, group_body, 0)

        pltpu.sync_copy(rwbuf, rw_hbm.at[pl.ds(base, CHUNK * E)])
        tbase = wid * (ROWS_PER_W * K) + c * (CHUNK * K)
        pltpu.sync_copy(tkibuf, tki_hbm.at[pl.ds(tbase, CHUNK * K)])
        pltpu.sync_copy(tkwbuf, tkw_hbm.at[pl.ds(tbase, CHUNK * K)])
        return 0

    lax.fori_loop(0, N_CHUNKS, chunk_body, 0)
    pltpu.sync_copy(cntbuf, pcnt_hbm.at[pl.ds(wid * E, E)])


@functools.partial(
    pl.kernel,
    out_type=(
        jax.ShapeDtypeStruct((B * E,), jnp.float32),
        jax.ShapeDtypeStruct((B * K,), jnp.int32),
        jax.ShapeDtypeStruct((B * K,), jnp.float32),
        jax.ShapeDtypeStruct((NW * E,), jnp.int32),
    ),
    mesh=plsc.VectorSubcoreMesh(core_axis_name="c", subcore_axis_name="s"),
    compiler_params=pltpu.CompilerParams(needs_layout_passes=False),
    scratch_types=[
        pltpu.VMEM((CHUNK * E,), jnp.float32),
        pltpu.VMEM((CHUNK * E,), jnp.float32),
        pltpu.VMEM((CHUNK * K,), jnp.int32),
        pltpu.VMEM((CHUNK * K,), jnp.float32),
        pltpu.VMEM((E,), jnp.int32),
    ],
)
def _router(gates_hbm, rw_hbm, tki_hbm, tkw_hbm, pcnt_hbm,
            gbuf, rwbuf, tkibuf, tkwbuf, cntbuf):
    _router_body(gates_hbm, rw_hbm, tki_hbm, tkw_hbm, pcnt_hbm,
                 gbuf, rwbuf, tkibuf, tkwbuf, cntbuf)


def _stats_body(pc_ref, loads_ref, util_ref, exc_ref):
    pc = pc_ref[...]
    loads = jnp.sum(pc, axis=0, keepdims=True)
    loads_ref[...] = loads
    util_ref[...] = loads.astype(jnp.float32) * (1.0 / B)
    exc_ref[...] = (loads > CAPACITY).astype(jnp.int32)


_stats = pl.pallas_call(
    _stats_body,
    out_shape=(
        jax.ShapeDtypeStruct((1, E), jnp.int32),
        jax.ShapeDtypeStruct((1, E), jnp.float32),
        jax.ShapeDtypeStruct((1, E), jnp.int32),
    ),
)


def kernel(gates):
    rw, tki, tkw, pcnt = _router(gates.reshape(-1))
    loads, util, exc = _stats(pcnt.reshape(NW, E))
    return (
        rw.reshape(B, E),
        tki.reshape(B, K),
        util.reshape(E),
        loads.reshape(E),
        exc.reshape(E).astype(jnp.bool_),
        tkw.reshape(B, K),
    )


# speedup vs baseline: 7.9042x; 1.0683x over previous
import functools

import jax
import jax.numpy as jnp
from jax import lax
from jax.experimental import pallas as pl
from jax.experimental.pallas import tpu as pltpu
from jax.experimental.pallas import tpu_sc as plsc

B = 32768
E = 64
EP = E + 1
K = 8
NC, NS = 2, 16
NW = NC * NS
ROWS_PER_W = B // NW
CHUNK = 256
N_CHUNKS = ROWS_PER_W // CHUNK
GROUPS = CHUNK // 16
CAPACITY = int(B * 1.25 / E)

_NET19 = [(0, 1), (2, 3), (4, 5), (6, 7),
          (0, 2), (1, 3), (4, 6), (5, 7),
          (1, 2), (5, 6),
          (0, 4), (1, 5), (2, 6), (3, 7),
          (2, 4), (3, 5),
          (1, 2), (3, 4), (5, 6)]
_BITONIC12 = [(0, 4), (1, 5), (2, 6), (3, 7),
              (0, 2), (1, 3), (4, 6), (5, 7),
              (0, 1), (2, 3), (4, 5), (6, 7)]


def _router_body(gates_hbm, rw_hbm, tki_hbm, tkw_hbm, pcnt_hbm,
                 gbuf, gbufp, rwbuf, tkibuf, tkwbuf, cntbuf):
    wid = lax.axis_index("s") * NC + lax.axis_index("c")
    lane = lax.iota(jnp.int32, 16)
    zf = jnp.zeros((16,), jnp.float32)
    zi = jnp.zeros((16,), jnp.int32)

    for q in range(E // 16):
        cntbuf[pl.ds(q * 16, 16)] = zi

    def chunk_body(c, _):
        base = wid * (ROWS_PER_W * E) + c * (CHUNK * E)
        pltpu.sync_copy(gates_hbm.at[pl.ds(base, CHUNK * E)], gbuf)

        def group_body(g, _):
            goff = g * (16 * E)
            goffp = g * (16 * EP)
            for r in range(16):
                for q in range(E // 16):
                    gbufp[pl.ds(goffp + r * EP + q * 16, 16)] = (
                        gbuf[pl.ds(goff + r * E + q * 16, 16)])
            for r in range(16):
                for q in range(E // 16):
                    rwbuf[pl.ds(goff + r * E + q * 16, 16)] = zf

            rowaddr = goff + lane * E
            rowaddrp = goffp + lane * EP

            def keys_of(t):
                ks = []
                for j in range(8):
                    e = t * 8 + j
                    v = plsc.load_gather(gbufp, [rowaddrp + e])
                    bits = lax.bitcast_convert_type(v, jnp.int32)
                    kb = (bits & -64) | (63 - e)
                    ks.append(lax.bitcast_convert_type(kb, jnp.float32))
                return ks

            def sort8(v):
                for i, j in _NET19:
                    v[i], v[j] = jnp.maximum(v[i], v[j]), jnp.minimum(v[i], v[j])
                return v

            def merge_top8(a, b):
                c8 = [jnp.maximum(a[i], b[7 - i]) for i in range(8)]
                for i, j in _BITONIC12:
                    c8[i], c8[j] = (jnp.maximum(c8[i], c8[j]),
                                    jnp.minimum(c8[i], c8[j]))
                return c8

            m01 = merge_top8(sort8(keys_of(0)), sort8(keys_of(1)))
            m23 = merge_top8(sort8(keys_of(2)), sort8(keys_of(3)))
            m03 = merge_top8(m01, m23)
            m45 = merge_top8(sort8(keys_of(4)), sort8(keys_of(5)))
            m67 = merge_top8(sort8(keys_of(6)), sort8(keys_of(7)))
            m47 = merge_top8(m45, m67)
            f = merge_top8(m03, m47)

            mi = [63 - (lax.bitcast_convert_type(f[j], jnp.int32) & 63)
                  for j in range(K)]
            m = [plsc.load_gather(gbufp, [rowaddrp + mi[j]]) for j in range(K)]

            p = [jnp.exp(m[j] - m[0]) for j in range(K)]
            s = p[0]
            for j in range(1, K):
                s = s + p[j]
            inv = 1.0 / s
            w = [p[j] * inv for j in range(K)]

            trow = g * (16 * K) + lane * K
            one = jnp.ones((16,), jnp.int32)
            for j in range(K):
                plsc.store_scatter(rwbuf, [rowaddr + mi[j]], w[j])
                plsc.store_scatter(tkwbuf, [trow + j], w[j])
                plsc.store_scatter(tkibuf, [trow + j], mi[j])
                plsc.addupdate_scatter(
                    cntbuf, [mi[j]], jnp.where(w[j] > 0.0, one, zi))
            return 0

        lax.fori_loop(0, GROUPS, group_body, 0)

        pltpu.sync_copy(rwbuf, rw_hbm.at[pl.ds(base, CHUNK * E)])
        tbase = wid * (ROWS_PER_W * K) + c * (CHUNK * K)
        pltpu.sync_copy(tkibuf, tki_hbm.at[pl.ds(tbase, CHUNK * K)])
        pltpu.sync_copy(tkwbuf, tkw_hbm.at[pl.ds(tbase, CHUNK * K)])
        return 0

    lax.fori_loop(0, N_CHUNKS, chunk_body, 0)
    pltpu.sync_copy(cntbuf, pcnt_hbm.at[pl.ds(wid * E, E)])


@functools.partial(
    pl.kernel,
    out_type=(
        jax.ShapeDtypeStruct((B * E,), jnp.float32),
        jax.ShapeDtypeStruct((B * K,), jnp.int32),
        jax.ShapeDtypeStruct((B * K,), jnp.float32),
        jax.ShapeDtypeStruct((NW * E,), jnp.int32),
    ),
    mesh=plsc.VectorSubcoreMesh(core_axis_name="c", subcore_axis_name="s"),
    compiler_params=pltpu.CompilerParams(needs_layout_passes=False),
    scratch_types=[
        pltpu.VMEM((CHUNK * E,), jnp.float32),
        pltpu.VMEM((CHUNK * EP,), jnp.float32),
        pltpu.VMEM((CHUNK * E,), jnp.float32),
        pltpu.VMEM((CHUNK * K,), jnp.int32),
        pltpu.VMEM((CHUNK * K,), jnp.float32),
        pltpu.VMEM((E,), jnp.int32),
    ],
)
def _router(gates_hbm, rw_hbm, tki_hbm, tkw_hbm, pcnt_hbm,
            gbuf, gbufp, rwbuf, tkibuf, tkwbuf, cntbuf):
    _router_body(gates_hbm, rw_hbm, tki_hbm, tkw_hbm, pcnt_hbm,
                 gbuf, gbufp, rwbuf, tkibuf, tkwbuf, cntbuf)


def _stats_body(pc_ref, loads_ref, util_ref, exc_ref):
    pc = pc_ref[...]
    loads = jnp.sum(pc, axis=0, keepdims=True)
    loads_ref[...] = loads
    util_ref[...] = loads.astype(jnp.float32) * (1.0 / B)
    exc_ref[...] = (loads > CAPACITY).astype(jnp.int32)


_stats = pl.pallas_call(
    _stats_body,
    out_shape=(
        jax.ShapeDtypeStruct((1, E), jnp.int32),
        jax.ShapeDtypeStruct((1, E), jnp.float32),
        jax.ShapeDtypeStruct((1, E), jnp.int32),
    ),
)


def kernel(gates):
    rw, tki, tkw, pcnt = _router(gates.reshape(-1))
    loads, util, exc = _stats(pcnt.reshape(NW, E))
    return (
        rw.reshape(B, E),
        tki.reshape(B, K),
        util.reshape(E),
        loads.reshape(E),
        exc.reshape(E).astype(jnp.bool_),
        tkw.reshape(B, K),
    )


# speedup vs baseline: 7.9320x; 1.0035x over previous
import functools

import jax
import jax.numpy as jnp
from jax import lax
from jax.experimental import pallas as pl
from jax.experimental.pallas import tpu as pltpu
from jax.experimental.pallas import tpu_sc as plsc

B = 32768
E = 64
EP = E + 1
K = 8
NC, NS = 2, 16
NW = NC * NS
ROWS_PER_W = B // NW
CHUNK = 256
N_CHUNKS = ROWS_PER_W // CHUNK
GROUPS = CHUNK // 16
CAPACITY = int(B * 1.25 / E)

_NET19 = [(0, 1), (2, 3), (4, 5), (6, 7),
          (0, 2), (1, 3), (4, 6), (5, 7),
          (1, 2), (5, 6),
          (0, 4), (1, 5), (2, 6), (3, 7),
          (2, 4), (3, 5),
          (1, 2), (3, 4), (5, 6)]
_BITONIC12 = [(0, 4), (1, 5), (2, 6), (3, 7),
              (0, 2), (1, 3), (4, 6), (5, 7),
              (0, 1), (2, 3), (4, 5), (6, 7)]


def _router_body(gates_hbm, rw_hbm, tki_hbm, tkw_hbm, pcnt_hbm,
                 gbuf, gbufp, rwbuf, tkibuf, tkwbuf, cntbuf):
    wid = lax.axis_index("s") * NC + lax.axis_index("c")
    lane = lax.iota(jnp.int32, 16)
    zf = jnp.zeros((16,), jnp.float32)
    zi = jnp.zeros((16,), jnp.int32)
    colbase = (lane & 1) * E
    lane8 = lane * K

    for q in range(E // 16):
        cntbuf[pl.ds(q * 16, 16)] = zi

    def chunk_body(c, _):
        hrow = wid * (ROWS_PER_W // 2) + c * (CHUNK // 2)
        pltpu.sync_copy(gates_hbm.at[pl.ds(hrow, CHUNK // 2), :], gbuf)

        def group_body(g, _):
            goffp = g * (16 * EP)
            for r in range(16):
                for q in range(E // 16):
                    gbufp[pl.ds(goffp + r * EP + q * 16, 16)] = (
                        gbuf[g * 8 + r // 2, pl.ds((r % 2) * E + q * 16, 16)])
            for r in range(16):
                for q in range(E // 16):
                    rwbuf[g * 8 + r // 2, pl.ds((r % 2) * E + q * 16, 16)] = zf

            rowaddrp = goffp + lane * EP
            row2 = lax.shift_right_logical(g * 16 + lane, 1)
            gvec = jnp.full((16,), g, jnp.int32)

            def keys_of(t):
                ks = []
                for j in range(8):
                    e = t * 8 + j
                    v = plsc.load_gather(gbufp, [rowaddrp + e])
                    bits = lax.bitcast_convert_type(v, jnp.int32)
                    kb = (bits & -64) | (63 - e)
                    ks.append(lax.bitcast_convert_type(kb, jnp.float32))
                return ks

            def sort8(v):
                for i, j in _NET19:
                    v[i], v[j] = jnp.maximum(v[i], v[j]), jnp.minimum(v[i], v[j])
                return v

            def merge_top8(a, b):
                c8 = [jnp.maximum(a[i], b[7 - i]) for i in range(8)]
                for i, j in _BITONIC12:
                    c8[i], c8[j] = (jnp.maximum(c8[i], c8[j]),
                                    jnp.minimum(c8[i], c8[j]))
                return c8

            m01 = merge_top8(sort8(keys_of(0)), sort8(keys_of(1)))
            m23 = merge_top8(sort8(keys_of(2)), sort8(keys_of(3)))
            m03 = merge_top8(m01, m23)
            m45 = merge_top8(sort8(keys_of(4)), sort8(keys_of(5)))
            m67 = merge_top8(sort8(keys_of(6)), sort8(keys_of(7)))
            m47 = merge_top8(m45, m67)
            f = merge_top8(m03, m47)

            mi = [63 - (lax.bitcast_convert_type(f[j], jnp.int32) & 63)
                  for j in range(K)]
            m = [plsc.load_gather(gbufp, [rowaddrp + mi[j]]) for j in range(K)]

            p = [jnp.exp(m[j] - m[0]) for j in range(K)]
            s = p[0]
            for j in range(1, K):
                s = s + p[j]
            inv = 1.0 / s
            w = [p[j] * inv for j in range(K)]

            one = jnp.ones((16,), jnp.int32)
            for j in range(K):
                plsc.store_scatter(rwbuf, [row2, colbase + mi[j]], w[j])
                plsc.store_scatter(tkwbuf, [gvec, lane8 + j], w[j])
                plsc.store_scatter(tkibuf, [gvec, lane8 + j], mi[j])
                plsc.addupdate_scatter(
                    cntbuf, [mi[j]], jnp.where(w[j] > 0.0, one, zi))
            return 0

        lax.fori_loop(0, GROUPS, group_body, 0)

        pltpu.sync_copy(rwbuf, rw_hbm.at[pl.ds(hrow, CHUNK // 2), :])
        trow = wid * (ROWS_PER_W * K // 128) + c * (CHUNK * K // 128)
        pltpu.sync_copy(tkibuf, tki_hbm.at[pl.ds(trow, CHUNK * K // 128), :])
        pltpu.sync_copy(tkwbuf, tkw_hbm.at[pl.ds(trow, CHUNK * K // 128), :])
        return 0

    lax.fori_loop(0, N_CHUNKS, chunk_body, 0)
    pltpu.sync_copy(cntbuf, pcnt_hbm.at[pl.ds(wid * E, E)])


@functools.partial(
    pl.kernel,
    out_type=(
        jax.ShapeDtypeStruct((B // 2, 2 * E), jnp.float32),
        jax.ShapeDtypeStruct((B * K // 128, 128), jnp.int32),
        jax.ShapeDtypeStruct((B * K // 128, 128), jnp.float32),
        jax.ShapeDtypeStruct((NW * E,), jnp.int32),
    ),
    mesh=plsc.VectorSubcoreMesh(core_axis_name="c", subcore_axis_name="s"),
    compiler_params=pltpu.CompilerParams(needs_layout_passes=False,
                                         use_tc_tiling_on_sc=True),
    scratch_types=[
        pltpu.VMEM((CHUNK // 2, 2 * E), jnp.float32),
        pltpu.VMEM((CHUNK * EP,), jnp.float32),
        pltpu.VMEM((CHUNK // 2, 2 * E), jnp.float32),
        pltpu.VMEM((CHUNK * K // 128, 128), jnp.int32),
        pltpu.VMEM((CHUNK * K // 128, 128), jnp.float32),
        pltpu.VMEM((E,), jnp.int32),
    ],
)
def _router(gates_hbm, rw_hbm, tki_hbm, tkw_hbm, pcnt_hbm,
            gbuf, gbufp, rwbuf, tkibuf, tkwbuf, cntbuf):
    _router_body(gates_hbm, rw_hbm, tki_hbm, tkw_hbm, pcnt_hbm,
                 gbuf, gbufp, rwbuf, tkibuf, tkwbuf, cntbuf)


def _stats_body(pc_ref, loads_ref, util_ref, exc_ref):
    pc = pc_ref[...]
    loads = jnp.sum(pc, axis=0, keepdims=True)
    loads_ref[...] = loads
    util_ref[...] = loads.astype(jnp.float32) * (1.0 / B)
    exc_ref[...] = (loads > CAPACITY).astype(jnp.int32)


_stats = pl.pallas_call(
    _stats_body,
    out_shape=(
        jax.ShapeDtypeStruct((1, E), jnp.int32),
        jax.ShapeDtypeStruct((1, E), jnp.float32),
        jax.ShapeDtypeStruct((1, E), jnp.int32),
    ),
)


def kernel(gates):
    rw2, tki2, tkw2, pcnt = _router(gates.reshape(B // 2, 2 * E))
    loads, util, exc = _stats(pcnt.reshape(NW, E))
    return (
        rw2.reshape(B, E),
        tki2.reshape(B, K),
        util.reshape(E),
        loads.reshape(E),
        exc.reshape(E).astype(jnp.bool_),
        tkw2.reshape(B, K),
    )
